# baseline (speedup 1.0000x reference)
import jax
import jax.numpy as jnp
from jax import lax
from jax.experimental import pallas as pl
from jax.experimental.pallas import tpu as pltpu
from jax.experimental.pallas import tpu_sc as plsc

_N_ENT = 100000
_N_REL = 1000
_D = 128
_TILE_ROWS = 3128
_CHUNK = 184
_NCHUNK = 17
_NBUF = 4
_LAG = 2
_REL_ROWS = 128


def _sc_copy_body(ent_in, rel_in, ent_out, rel_out,
                  buf0, buf1, buf2, buf3,
                  isem0, isem1, isem2, isem3,
                  osem0, osem1, osem2, osem3,
                  relbuf, rsem):
    bufs = (buf0, buf1, buf2, buf3)
    isems = (isem0, isem1, isem2, isem3)
    osems = (osem0, osem1, osem2, osem3)
    wid = lax.axis_index("s") * 2 + lax.axis_index("c")
    base = jnp.minimum(wid * _TILE_ROWS, _N_ENT - _TILE_ROWS)

    @pl.when(wid < _N_REL // _REL_ROWS + (1 if _N_REL % _REL_ROWS else 0))
    def _():
        rbase = jnp.minimum(wid * _REL_ROWS, _N_REL - _REL_ROWS)
        cin = pltpu.make_async_copy(
            rel_in.at[pl.ds(rbase, _REL_ROWS)], relbuf, rsem)
        cin.start()
        cin.wait()
        cout = pltpu.make_async_copy(
            relbuf, rel_out.at[pl.ds(rbase, _REL_ROWS)], rsem)
        cout.start()
        cout.wait()

    in_c = [None] * _NCHUNK
    out_c = [None] * _NCHUNK
    for i in range(_NCHUNK + _LAG):
        if i < _NCHUNK:
            slot = i % _NBUF
            if i >= _NBUF:
                out_c[i - _NBUF].wait()
            c = pltpu.make_async_copy(
                ent_in.at[pl.ds(base + i * _CHUNK, _CHUNK)],
                bufs[slot], isems[slot])
            c.start()
            in_c[i] = c
        j = i - _LAG
        if j >= 0:
            in_c[j].wait()
            c = pltpu.make_async_copy(
                bufs[j % _NBUF],
                ent_out.at[pl.ds(base + j * _CHUNK, _CHUNK)],
                osems[j % _NBUF])
            c.start()
            out_c[j] = c
    for j in range(_NCHUNK - _NBUF, _NCHUNK):
        out_c[j].wait()


def kernel(entity_weight, relation_weight):
    mesh = plsc.VectorSubcoreMesh(core_axis_name="c", subcore_axis_name="s")
    run = pl.kernel(
        _sc_copy_body,
        out_type=[
            jax.ShapeDtypeStruct((_N_ENT, _D), jnp.float32),
            jax.ShapeDtypeStruct((_N_REL, _D), jnp.float32),
        ],
        mesh=mesh,
        scratch_types=[
            pltpu.VMEM((_CHUNK, _D), jnp.float32),
            pltpu.VMEM((_CHUNK, _D), jnp.float32),
            pltpu.VMEM((_CHUNK, _D), jnp.float32),
            pltpu.VMEM((_CHUNK, _D), jnp.float32),
            pltpu.SemaphoreType.DMA,
            pltpu.SemaphoreType.DMA,
            pltpu.SemaphoreType.DMA,
            pltpu.SemaphoreType.DMA,
            pltpu.SemaphoreType.DMA,
            pltpu.SemaphoreType.DMA,
            pltpu.SemaphoreType.DMA,
            pltpu.SemaphoreType.DMA,
            pltpu.VMEM((_REL_ROWS, _D), jnp.float32),
            pltpu.SemaphoreType.DMA,
        ],
    )
    ent_out, rel_out = run(entity_weight, relation_weight)
    return (ent_out, rel_out)


# speedup vs baseline: 1.2033x; 1.2033x over previous
import jax
import jax.numpy as jnp
from jax import lax
from jax.experimental import pallas as pl
from jax.experimental.pallas import tpu as pltpu
from jax.experimental.pallas import tpu_sc as plsc

_ENT_BLOCK = 25000
_N_REL = 1000
_D = 128
_REL_ROWS = 128


def _tc_ent_body(ent_in, ent_out):
    ent_out[...] = ent_in[...]


def _sc_rel_body(rel_in, rel_out, relbuf, rsem):
    wid = lax.axis_index("s") * 2 + lax.axis_index("c")

    @pl.when(wid < 8)
    def _():
        rbase = jnp.minimum(wid * _REL_ROWS, _N_REL - _REL_ROWS)
        cin = pltpu.make_async_copy(
            rel_in.at[pl.ds(rbase, _REL_ROWS)], relbuf, rsem)
        cin.start()
        cin.wait()
        cout = pltpu.make_async_copy(
            relbuf, rel_out.at[pl.ds(rbase, _REL_ROWS)], rsem)
        cout.start()
        cout.wait()


def kernel(entity_weight, relation_weight):
    n_ent, d = entity_weight.shape
    grid = n_ent // _ENT_BLOCK
    ent_out = pl.pallas_call(
        _tc_ent_body,
        grid=(grid,),
        in_specs=[pl.BlockSpec((_ENT_BLOCK, d), lambda i: (i, 0))],
        out_specs=pl.BlockSpec((_ENT_BLOCK, d), lambda i: (i, 0)),
        out_shape=jax.ShapeDtypeStruct(entity_weight.shape, entity_weight.dtype),
    )(entity_weight)

    rel_run = pl.kernel(
        _sc_rel_body,
        out_type=jax.ShapeDtypeStruct((_N_REL, _D), jnp.float32),
        mesh=plsc.VectorSubcoreMesh(core_axis_name="c", subcore_axis_name="s"),
        scratch_types=[
            pltpu.VMEM((_REL_ROWS, _D), jnp.float32),
            pltpu.SemaphoreType.DMA,
        ],
    )
    rel_out = rel_run(relation_weight)
    return (ent_out, rel_out)
